# Initial kernel scaffold; baseline (speedup 1.0000x reference)
import jax
import jax.numpy as jnp
from jax.experimental import pallas as pl


def kernel(x, table):
    raise NotImplementedError("write your pallas kernel here")


# speedup vs baseline: 1.4787x; 1.4787x over previous
import functools

import jax
import jax.numpy as jnp
from jax import lax
from jax.experimental import pallas as pl
from jax.experimental.pallas import tpu as pltpu
from jax.experimental.pallas import tpu_sc as plsc

VOCAB = 1000000
EMBED_DIM = 32
BATCH = 4096
SEQ = 200

NC = 2
NS = 16
NW = NC * NS

TOTAL = BATCH * SEQ
B_PER_W = TOTAL // NW
CHUNK = 128
GATHERS_PER_SUPER = 8
SUPER = CHUNK * GATHERS_PER_SUPER
N_SUPER = B_PER_W // SUPER
N_CHUNKS = B_PER_W // CHUNK

_mesh = plsc.VectorSubcoreMesh(
    core_axis_name="c", subcore_axis_name="s", num_cores=NC, num_subcores=NS
)


@functools.partial(
    pl.kernel,
    out_type=jax.ShapeDtypeStruct((TOTAL, EMBED_DIM), jnp.float32),
    mesh=_mesh,
    scratch_types=[
        pltpu.VMEM((N_CHUNKS, CHUNK), jnp.int32),
        pltpu.VMEM((SUPER, EMBED_DIM), jnp.float32),
        pltpu.SemaphoreType.DMA,
    ],
    compiler_params=pltpu.CompilerParams(use_tc_tiling_on_sc=False),
)
def _embed_sc(idx_hbm, table_hbm, out_hbm, idx_v, rows_v, sem):
    wid = lax.axis_index("s") * NC + lax.axis_index("c")
    pltpu.sync_copy(idx_hbm.at[wid], idx_v)
    row_base = wid * B_PER_W

    @pl.loop(0, N_SUPER)
    def _super(s):
        copies = []
        for j in range(GATHERS_PER_SUPER):
            copies.append(
                pltpu.async_copy(
                    table_hbm.at[idx_v.at[s * GATHERS_PER_SUPER + j]],
                    rows_v.at[pl.ds(j * CHUNK, CHUNK)],
                    sem,
                )
            )
        for c in copies:
            c.wait()
        pltpu.sync_copy(rows_v, out_hbm.at[pl.ds(row_base + s * SUPER, SUPER)])


def kernel(x, table):
    idx = x.astype(jnp.int32).reshape(NW, N_CHUNKS, CHUNK)
    out = _embed_sc(idx, table)
    return out.reshape(BATCH, SEQ, EMBED_DIM)


# speedup vs baseline: 1.4943x; 1.0105x over previous
import functools

import jax
import jax.numpy as jnp
from jax import lax
from jax.experimental import pallas as pl
from jax.experimental.pallas import tpu as pltpu
from jax.experimental.pallas import tpu_sc as plsc

VOCAB = 1000000
EMBED_DIM = 32
BATCH = 4096
SEQ = 200

NC = 2
NS = 16
NW = NC * NS

TOTAL = BATCH * SEQ
B_PER_W = TOTAL // NW
CHUNK = 128
GATHERS_PER_SUPER = 10
SUPER = CHUNK * GATHERS_PER_SUPER
N_SUPER = B_PER_W // SUPER
N_CHUNKS = B_PER_W // CHUNK

_mesh = plsc.VectorSubcoreMesh(
    core_axis_name="c", subcore_axis_name="s", num_cores=NC, num_subcores=NS
)


@functools.partial(
    pl.kernel,
    out_type=jax.ShapeDtypeStruct((TOTAL, EMBED_DIM), jnp.float32),
    mesh=_mesh,
    scratch_types=[
        pltpu.VMEM((N_CHUNKS, CHUNK), jnp.int32),
        pltpu.VMEM((SUPER, EMBED_DIM), jnp.float32),
        pltpu.VMEM((SUPER, EMBED_DIM), jnp.float32),
        pltpu.SemaphoreType.DMA,
        pltpu.SemaphoreType.DMA,
        pltpu.SemaphoreType.DMA,
        pltpu.SemaphoreType.DMA,
    ],
    compiler_params=pltpu.CompilerParams(use_tc_tiling_on_sc=False),
)
def _embed_sc(idx_hbm, table_hbm, out_hbm, idx_v, rows0, rows1,
              sem_g0, sem_g1, sem_o0, sem_o1):
    wid = lax.axis_index("s") * NC + lax.axis_index("c")
    pltpu.sync_copy(idx_hbm.at[wid], idx_v)
    row_base = wid * B_PER_W

    def fire(cur, buf, sem):
        for j in range(GATHERS_PER_SUPER):
            pltpu.async_copy(
                table_hbm.at[idx_v.at[cur * GATHERS_PER_SUPER + j]],
                buf.at[pl.ds(j * CHUNK, CHUNK)],
                sem,
            )

    def drain_gathers(buf, sem):
        pltpu.make_async_copy(table_hbm.at[pl.ds(0, SUPER)], buf, sem).wait()

    def drain_copyout(buf, sem):
        pltpu.make_async_copy(table_hbm.at[pl.ds(0, SUPER)], buf, sem).wait()

    fire(0, rows0, sem_g0)

    @pl.loop(0, N_SUPER, step=2)
    def _pair(s):
        for off, buf, semg, obuf, osemg, osemo, semo in (
            (0, rows0, sem_g0, rows1, sem_g1, sem_o1, sem_o0),
            (1, rows1, sem_g1, rows0, sem_g0, sem_o0, sem_o1),
        ):
            cur = s + off
            drain_gathers(buf, semg)
            pltpu.async_copy(
                buf, out_hbm.at[pl.ds(row_base + cur * SUPER, SUPER)], semo
            )

            @pl.when(jnp.logical_and(cur + 1 < N_SUPER, cur >= 1))
            def _():
                drain_copyout(obuf, osemo)

            @pl.when(cur + 1 < N_SUPER)
            def _():
                fire(cur + 1, obuf, osemg)

    drain_copyout(rows0, sem_o0)
    drain_copyout(rows1, sem_o1)


def kernel(x, table):
    idx = x.astype(jnp.int32).reshape(NW, N_CHUNKS, CHUNK)
    out = _embed_sc(idx, table)
    return out.reshape(BATCH, SEQ, EMBED_DIM)


# speedup vs baseline: 1.5012x; 1.0046x over previous
import functools

import jax
import jax.numpy as jnp
from jax import lax
from jax.experimental import pallas as pl
from jax.experimental.pallas import tpu as pltpu
from jax.experimental.pallas import tpu_sc as plsc

VOCAB = 1000000
EMBED_DIM = 32
BATCH = 4096
SEQ = 200

NC = 2
NS = 16
NW = NC * NS

TOTAL = BATCH * SEQ
B_PER_W = TOTAL // NW
CHUNK = 128
GATHERS_PER_SUPER = 10
SUPER = CHUNK * GATHERS_PER_SUPER
N_SUPER = B_PER_W // SUPER
N_CHUNKS = B_PER_W // CHUNK

_mesh = plsc.VectorSubcoreMesh(
    core_axis_name="c", subcore_axis_name="s", num_cores=NC, num_subcores=NS
)


@functools.partial(
    pl.kernel,
    out_type=jax.ShapeDtypeStruct((TOTAL, EMBED_DIM), jnp.float32),
    mesh=_mesh,
    scratch_types=[
        pltpu.VMEM((N_CHUNKS, CHUNK), jnp.int32),
        pltpu.VMEM((SUPER, EMBED_DIM), jnp.float32),
        pltpu.VMEM((SUPER, EMBED_DIM), jnp.float32),
        pltpu.SemaphoreType.DMA,
        pltpu.SemaphoreType.DMA,
        pltpu.SemaphoreType.DMA,
        pltpu.SemaphoreType.DMA,
    ],
    compiler_params=pltpu.CompilerParams(use_tc_tiling_on_sc=False),
)
def _embed_sc(idx_hbm, table_hbm, out_hbm, idx_v, rows0, rows1,
              sem_g0, sem_g1, sem_o0, sem_o1):
    wid = lax.axis_index("s") * NC + lax.axis_index("c")
    pltpu.sync_copy(idx_hbm.at[wid], idx_v)
    row_base = wid * B_PER_W

    def fire(cur, buf, sem):
        for j in range(GATHERS_PER_SUPER):
            pltpu.async_copy(
                table_hbm.at[idx_v.at[cur * GATHERS_PER_SUPER + j]],
                buf.at[pl.ds(j * CHUNK, CHUNK)],
                sem,
            )

    def drain_gathers(buf, sem):
        pltpu.make_async_copy(table_hbm.at[pl.ds(0, SUPER)], buf, sem).wait()

    def drain_copyout(buf, sem):
        pltpu.make_async_copy(table_hbm.at[pl.ds(0, SUPER)], buf, sem).wait()

    fire(0, rows0, sem_g0)

    @pl.loop(0, N_SUPER, step=2)
    def _pair(s):
        for off, buf, semg, obuf, osemg, osemo, semo in (
            (0, rows0, sem_g0, rows1, sem_g1, sem_o1, sem_o0),
            (1, rows1, sem_g1, rows0, sem_g0, sem_o0, sem_o1),
        ):
            cur = s + off

            @pl.when(jnp.logical_and(cur + 1 < N_SUPER, cur >= 1))
            def _():
                drain_copyout(obuf, osemo)

            @pl.when(cur + 1 < N_SUPER)
            def _():
                fire(cur + 1, obuf, osemg)

            drain_gathers(buf, semg)
            pltpu.async_copy(
                buf, out_hbm.at[pl.ds(row_base + cur * SUPER, SUPER)], semo
            )

    drain_copyout(rows0, sem_o0)
    drain_copyout(rows1, sem_o1)


def kernel(x, table):
    idx = x.astype(jnp.int32).reshape(NW, N_CHUNKS, CHUNK)
    out = _embed_sc(idx, table)
    return out.reshape(BATCH, SEQ, EMBED_DIM)
